# Initial kernel scaffold; baseline (speedup 1.0000x reference)
import jax
import jax.numpy as jnp
from jax.experimental import pallas as pl


def kernel(x, condition_mask, embedding_net_id, condition_embedding):
    raise NotImplementedError("write your pallas kernel here")


# speedup vs baseline: 21.5270x; 21.5270x over previous
import jax
import jax.numpy as jnp
from jax.experimental import pallas as pl

_HID = 32
_GROUP = 3 * _HID
_BB = 256
_NN = 40


_DN = (((0,), (0,)), ((), ()))


def _tile_kernel(xt_ref, mt_ref, wx_ref, wm_ref, t_ref, o_ref):
    acc = jax.lax.dot_general(xt_ref[...], wx_ref[...], _DN,
                              preferred_element_type=jnp.float32)
    acc = acc + jax.lax.dot_general(mt_ref[...], wm_ref[...], _DN,
                                    preferred_element_type=jnp.float32)
    o_ref[...] = acc + t_ref[...]


def kernel(x, condition_mask, embedding_net_id, condition_embedding):
    B, N = x.shape
    H = embedding_net_id.shape[1]
    C = _NN * _GROUP
    ce = condition_embedding.reshape(H)

    c = jnp.arange(C)
    node_local = c // _GROUP
    j = c % _GROUP
    n_iota = jnp.arange(_NN)[:, None]
    wx = jnp.where((node_local[None, :] == n_iota) & (j[None, :] < H),
                   jnp.float32(1.0), jnp.float32(0.0))
    wm = jnp.where((node_local[None, :] == n_iota) & (j[None, :] >= 2 * H),
                   ce[jnp.clip(j - 2 * H, 0, H - 1)][None, :],
                   jnp.float32(0.0))

    tfull = (jnp.zeros((N, 3, H), jnp.float32)
             .at[:, 1, :].set(embedding_net_id)
             .reshape(1, N * _GROUP))

    xt = x.T
    mt = condition_mask.T

    grid = (B // _BB, N // _NN)

    out = pl.pallas_call(
        _tile_kernel,
        grid=grid,
        in_specs=[
            pl.BlockSpec((_NN, _BB), lambda bi, nj: (nj, bi)),
            pl.BlockSpec((_NN, _BB), lambda bi, nj: (nj, bi)),
            pl.BlockSpec((_NN, C), lambda bi, nj: (0, 0)),
            pl.BlockSpec((_NN, C), lambda bi, nj: (0, 0)),
            pl.BlockSpec((1, C), lambda bi, nj: (0, nj)),
        ],
        out_specs=pl.BlockSpec((_BB, C), lambda bi, nj: (bi, nj)),
        out_shape=jax.ShapeDtypeStruct((B, N * _GROUP), jnp.float32),
    )(xt, mt, wx, wm, tfull)
    return out
